# baseline (speedup 1.0000x reference)
import functools

import jax
import jax.numpy as jnp
from jax import lax
from jax.experimental import pallas as pl
from jax.experimental.pallas import tpu as pltpu
from jax.experimental.pallas import tpu_sc as plsc

N = 10000
E = 320000
D = 128

NTILES = 32
CB = 128
NCH = 80
GC = 8
NG = NCH // GC
NB = 2
EP = NTILES * NCH * CB
NPAD = 10240
NPT = NPAD // 16


_RB = 1000


def _tc1_body(x_ref, w1_ref, amat_ref, h1_ref, pq_ref):
    h1 = jnp.dot(x_ref[...], w1_ref[...], preferred_element_type=jnp.float32)
    h1_ref[...] = h1
    pq_ref[...] = jnp.dot(h1, amat_ref[...], preferred_element_type=jnp.float32)


def _tc1(x, W1, amat):
    return pl.pallas_call(
        _tc1_body,
        grid=(N // _RB,),
        in_specs=[
            pl.BlockSpec((_RB, D), lambda i: (i, 0)),
            pl.BlockSpec((D, D), lambda i: (0, 0)),
            pl.BlockSpec((D, D), lambda i: (0, 0)),
        ],
        out_specs=[
            pl.BlockSpec((_RB, D), lambda i: (i, 0)),
            pl.BlockSpec((_RB, D), lambda i: (i, 0)),
        ],
        out_shape=[
            jax.ShapeDtypeStruct((N, D), jnp.float32),
            jax.ShapeDtypeStruct((N, D), jnp.float32),
        ],
    )(x, W1, amat)


def _tc2_body(p0_ref, p1_ref, w2_ref, h2_ref):
    h = jnp.maximum(p0_ref[...] + p1_ref[...], 0.0)
    h2_ref[...] = jnp.dot(h, w2_ref[...], preferred_element_type=jnp.float32)


def _tc2(p0, p1, W2):
    return pl.pallas_call(
        _tc2_body,
        grid=(N // _RB,),
        in_specs=[
            pl.BlockSpec((_RB, D), lambda i: (i, 0)),
            pl.BlockSpec((_RB, D), lambda i: (i, 0)),
            pl.BlockSpec((D, D), lambda i: (0, 0)),
        ],
        out_specs=pl.BlockSpec((_RB, D), lambda i: (i, 0)),
        out_shape=jax.ShapeDtypeStruct((N, D), jnp.float32),
    )(p0, p1, W2)


def _tc3_body(p0_ref, p1_ref, x_ref, g_ref, b_ref, o_ref):
    h = jnp.maximum(p0_ref[...] + p1_ref[...], 0.0) + x_ref[...]
    mu = jnp.mean(h, axis=-1, keepdims=True)
    var = jnp.mean((h - mu) * (h - mu), axis=-1, keepdims=True)
    o_ref[...] = (h - mu) * lax.rsqrt(var + 1e-5) * g_ref[...] + b_ref[...]


def _tc3(p0, p1, x, gamma, beta):
    return pl.pallas_call(
        _tc3_body,
        grid=(N // _RB,),
        in_specs=[
            pl.BlockSpec((_RB, D), lambda i: (i, 0)),
            pl.BlockSpec((_RB, D), lambda i: (i, 0)),
            pl.BlockSpec((_RB, D), lambda i: (i, 0)),
            pl.BlockSpec((1, D), lambda i: (0, 0)),
            pl.BlockSpec((1, D), lambda i: (0, 0)),
        ],
        out_specs=pl.BlockSpec((_RB, D), lambda i: (i, 0)),
        out_shape=jax.ShapeDtypeStruct((N, D), jnp.float32),
    )(p0, p1, x, gamma, beta)


def _make_spmm(first):
    mesh = plsc.VectorSubcoreMesh(core_axis_name="c", subcore_axis_name="s")
    out_type = [jax.ShapeDtypeStruct((2, NPAD, D), jnp.float32)]
    if first:
        out_type.append(jax.ShapeDtypeStruct((NTILES, NCH, CB), jnp.float32))
    scratch = [
        pltpu.VMEM((2, GC, CB), jnp.int32),
        pltpu.VMEM((2, GC, CB), jnp.int32),
        pltpu.VMEM((2, GC, CB), jnp.float32),
        pltpu.VMEM((NB, CB, D), jnp.float32),
        pltpu.VMEM_SHARED((NPAD, D), jnp.float32),
        pltpu.SemaphoreType.DMA((NB,)),
        pltpu.SemaphoreType.DMA((NB,)),
    ]
    if first:
        scratch += [
            pltpu.VMEM((NB, CB), jnp.float32),
            pltpu.VMEM((NB, CB), jnp.float32),
        ]

    def body(h_hbm, row_hbm, col_hbm, wsrc_hbm, *rest):
        if first:
            (p_hbm, q_hbm, out_hbm, w_hbm,
             row_g, col_g, wv_g, rows_v, acc, sem, ssem, pv_v, qv_v) = rest
        else:
            out_hbm, row_g, col_g, wv_g, rows_v, acc, sem, ssem = rest
        c = lax.axis_index("c")
        s = lax.axis_index("s")
        wid = s * 2 + c

        def load_group(gidx, gpar):
            goff = gidx * GC
            pltpu.sync_copy(row_hbm.at[wid, pl.ds(goff, GC)], row_g.at[gpar])
            pltpu.sync_copy(col_hbm.at[wid, pl.ds(goff, GC)], col_g.at[gpar])
            pltpu.sync_copy(wsrc_hbm.at[wid, pl.ds(goff, GC)], wv_g.at[gpar])

        def issue_gathers(gpar, jg, par):
            pltpu.async_copy(h_hbm.at[col_g.at[gpar, jg]], rows_v.at[par],
                             sem.at[par])
            if first:
                pltpu.async_copy(p_hbm.at[row_g.at[gpar, jg]], pv_v.at[par],
                                 sem.at[par])
                pltpu.async_copy(q_hbm.at[col_g.at[gpar, jg]], qv_v.at[par],
                                 sem.at[par])

        def wait_gathers(gpar, jg, par):
            pltpu.make_async_copy(h_hbm.at[col_g.at[gpar, jg]],
                                  rows_v.at[par], sem.at[par]).wait()
            if first:
                pltpu.make_async_copy(p_hbm.at[row_g.at[gpar, jg]],
                                      pv_v.at[par], sem.at[par]).wait()
                pltpu.make_async_copy(q_hbm.at[col_g.at[gpar, jg]],
                                      qv_v.at[par], sem.at[par]).wait()

        zero16 = jnp.zeros((16,), jnp.float32)

        def zbody(r, carry):
            for v in range(8):
                rows_v[0, r, pl.ds(v * 16, 16)] = zero16
            return carry

        lax.fori_loop(0, CB, zbody, 0)
        base = s * NPT
        for k in range(NPT // CB):
            pltpu.sync_copy(rows_v.at[0], acc.at[pl.ds(base + k * CB, CB)])
        plsc.subcore_barrier()

        load_group(0, 0)
        for j0 in range(NB - 1):
            issue_gathers(0, j0, j0)

        def jbody(j, carry):
            jg = j % GC
            gidx = j // GC
            gpar = gidx % 2
            par = j % NB
            nj = j + NB - 1
            njg = nj % GC
            ngidx = nj // GC
            ngpar = ngidx % 2
            npar = nj % NB

            @pl.when(jnp.logical_and(nj < NCH, njg == 0))
            def _():
                load_group(ngidx, ngpar)

            @pl.when(nj < NCH)
            def _():
                @pl.when(nj >= NB)
                def _():
                    pltpu.make_async_copy(
                        rows_v.at[npar], acc.at[row_g.at[gpar, jg]],
                        ssem.at[npar]).wait()

                issue_gathers(ngpar, njg, npar)

            wait_gathers(gpar, jg, par)

            if first:
                for g in range(CB // 16):
                    sl = pl.ds(g * 16, 16)
                    sv = pv_v[par, sl] + qv_v[par, sl]
                    sv = jnp.where(sv > 0.0, sv, 0.2 * sv)
                    att = 1.0 / (1.0 + jnp.exp(-sv))
                    wv_g[gpar, jg, sl] = att * wv_g[gpar, jg, sl]

            @plsc.parallel_loop(0, CB // 16, unroll=4)
            def scale(k16):
                w16 = wv_g[gpar, jg, pl.ds(k16 * 16, 16)]
                for e in range(16):
                    wb = jnp.full((16,), w16[e])
                    r = k16 * 16 + e
                    for v in range(8):
                        sl = pl.ds(v * 16, 16)
                        rows_v[par, r, sl] = rows_v[par, r, sl] * wb

            pltpu.async_copy(rows_v.at[par], acc.at[row_g.at[gpar, jg]],
                             ssem.at[par], add=True)
            if first:
                @pl.when(jg == GC - 1)
                def _():
                    pltpu.sync_copy(wv_g.at[gpar],
                                    w_hbm.at[wid, pl.ds(gidx * GC, GC)])
            return carry

        lax.fori_loop(0, NCH, jbody, 0)
        for k in range(NB):
            pltpu.make_async_copy(rows_v.at[k], acc.at[row_g.at[0, 0]],
                                  ssem.at[k]).wait()
        plsc.subcore_barrier()
        pltpu.sync_copy(acc.at[pl.ds(base, NPT)], out_hbm.at[c, pl.ds(base, NPT)])

    return pl.kernel(body, mesh=mesh, out_type=out_type,
                     scratch_types=scratch,
                     compiler_params=pltpu.CompilerParams(
                         needs_layout_passes=False))


def kernel(x, edge_index, edge_values, W1, a1, W2, ln_gamma, ln_beta):
    row = edge_index[0]
    col = edge_index[1]
    pad = EP - E
    spread = (jnp.arange(pad, dtype=jnp.int32) * 37) % N
    rowp = jnp.concatenate([row, spread]).reshape(NTILES, NCH, CB)
    colp = jnp.concatenate([col, spread]).reshape(NTILES, NCH, CB)
    evp = jnp.concatenate([edge_values, jnp.zeros((pad,), jnp.float32)]).reshape(NTILES, NCH, CB)

    amat = jnp.zeros((D, D), jnp.float32)
    amat = amat.at[:, 0].set(a1[:D, 0]).at[:, 1].set(a1[D:, 0])

    h1, pq = _tc1(x, W1, amat)
    p = pq[:, 0]
    q = pq[:, 1]

    parts1, w = _make_spmm(True)(h1, rowp, colp, evp, p, q)

    h2 = _tc2(parts1[0], parts1[1], W2)

    (parts2,) = _make_spmm(False)(h2, rowp, colp, w)

    out = _tc3(parts2[0], parts2[1], x, ln_gamma.reshape(1, D),
               ln_beta.reshape(1, D))
    return out


# speedup vs baseline: 2.1992x; 2.1992x over previous
import functools

import jax
import jax.numpy as jnp
from jax import lax
from jax.experimental import pallas as pl
from jax.experimental.pallas import tpu as pltpu
from jax.experimental.pallas import tpu_sc as plsc

N = 10000
E = 320000
D = 128

NTILES = 32
CB = 64
NCH = 160
GC = 16
NG = NCH // GC
NB = 4
EP = NTILES * NCH * CB
NPAD = 10240
NPT = NPAD // 16


_RB = 1000


def _tc1_body(x_ref, w1_ref, amat_ref, h1_ref, pq_ref):
    h1 = jnp.dot(x_ref[...], w1_ref[...], preferred_element_type=jnp.float32)
    h1_ref[...] = h1
    pq_ref[...] = jnp.dot(h1, amat_ref[...], preferred_element_type=jnp.float32)


def _tc1(x, W1, amat):
    return pl.pallas_call(
        _tc1_body,
        grid=(N // _RB,),
        in_specs=[
            pl.BlockSpec((_RB, D), lambda i: (i, 0)),
            pl.BlockSpec((D, D), lambda i: (0, 0)),
            pl.BlockSpec((D, D), lambda i: (0, 0)),
        ],
        out_specs=[
            pl.BlockSpec((_RB, D), lambda i: (i, 0)),
            pl.BlockSpec((_RB, D), lambda i: (i, 0)),
        ],
        out_shape=[
            jax.ShapeDtypeStruct((N, D), jnp.float32),
            jax.ShapeDtypeStruct((N, D), jnp.float32),
        ],
    )(x, W1, amat)


def _tc2_body(p0_ref, p1_ref, w2_ref, h2_ref):
    h = jnp.maximum(p0_ref[...] + p1_ref[...], 0.0)
    h2_ref[...] = jnp.dot(h, w2_ref[...], preferred_element_type=jnp.float32)


def _tc2(p0, p1, W2):
    return pl.pallas_call(
        _tc2_body,
        grid=(N // _RB,),
        in_specs=[
            pl.BlockSpec((_RB, D), lambda i: (i, 0)),
            pl.BlockSpec((_RB, D), lambda i: (i, 0)),
            pl.BlockSpec((D, D), lambda i: (0, 0)),
        ],
        out_specs=pl.BlockSpec((_RB, D), lambda i: (i, 0)),
        out_shape=jax.ShapeDtypeStruct((N, D), jnp.float32),
    )(p0, p1, W2)


def _tc3_body(p0_ref, p1_ref, x_ref, g_ref, b_ref, o_ref):
    h = jnp.maximum(p0_ref[...] + p1_ref[...], 0.0) + x_ref[...]
    mu = jnp.mean(h, axis=-1, keepdims=True)
    var = jnp.mean((h - mu) * (h - mu), axis=-1, keepdims=True)
    o_ref[...] = (h - mu) * lax.rsqrt(var + 1e-5) * g_ref[...] + b_ref[...]


def _tc3(p0, p1, x, gamma, beta):
    return pl.pallas_call(
        _tc3_body,
        grid=(N // _RB,),
        in_specs=[
            pl.BlockSpec((_RB, D), lambda i: (i, 0)),
            pl.BlockSpec((_RB, D), lambda i: (i, 0)),
            pl.BlockSpec((_RB, D), lambda i: (i, 0)),
            pl.BlockSpec((1, D), lambda i: (0, 0)),
            pl.BlockSpec((1, D), lambda i: (0, 0)),
        ],
        out_specs=pl.BlockSpec((_RB, D), lambda i: (i, 0)),
        out_shape=jax.ShapeDtypeStruct((N, D), jnp.float32),
    )(p0, p1, x, gamma, beta)


def _make_spmm(first):
    mesh = plsc.VectorSubcoreMesh(core_axis_name="c", subcore_axis_name="s")
    out_type = [jax.ShapeDtypeStruct((2, NPAD, D), jnp.float32)]
    if first:
        out_type.append(jax.ShapeDtypeStruct((NTILES, NCH, CB), jnp.float32))
    scratch = [
        pltpu.VMEM((2, GC, CB), jnp.int32),
        pltpu.VMEM((2, GC, CB), jnp.int32),
        pltpu.VMEM((2, GC, CB), jnp.float32),
        pltpu.VMEM((NB, CB, D), jnp.float32),
        pltpu.VMEM_SHARED((NPAD, D), jnp.float32),
        pltpu.SemaphoreType.DMA((NB,)),
        pltpu.SemaphoreType.DMA((NB,)),
    ]
    if first:
        scratch += [
            pltpu.VMEM((NB, CB), jnp.float32),
            pltpu.VMEM((NB, CB), jnp.float32),
        ]

    def body(h_hbm, row_hbm, col_hbm, wsrc_hbm, *rest):
        if first:
            (p_hbm, q_hbm, out_hbm, w_hbm,
             row_g, col_g, wv_g, rows_v, acc, sem, ssem, pv_v, qv_v) = rest
        else:
            out_hbm, row_g, col_g, wv_g, rows_v, acc, sem, ssem = rest
        c = lax.axis_index("c")
        s = lax.axis_index("s")
        wid = s * 2 + c

        def load_group(gidx, gpar):
            goff = gidx * GC
            pltpu.sync_copy(row_hbm.at[wid, pl.ds(goff, GC)], row_g.at[gpar])
            pltpu.sync_copy(col_hbm.at[wid, pl.ds(goff, GC)], col_g.at[gpar])
            pltpu.sync_copy(wsrc_hbm.at[wid, pl.ds(goff, GC)], wv_g.at[gpar])

        def issue_gathers(gpar, jg, par):
            pltpu.async_copy(h_hbm.at[col_g.at[gpar, jg]], rows_v.at[par],
                             sem.at[par])
            if first:
                pltpu.async_copy(p_hbm.at[row_g.at[gpar, jg]], pv_v.at[par],
                                 sem.at[par])
                pltpu.async_copy(q_hbm.at[col_g.at[gpar, jg]], qv_v.at[par],
                                 sem.at[par])

        def wait_gathers(gpar, jg, par):
            pltpu.make_async_copy(h_hbm.at[col_g.at[gpar, jg]],
                                  rows_v.at[par], sem.at[par]).wait()
            if first:
                pltpu.make_async_copy(p_hbm.at[row_g.at[gpar, jg]],
                                      pv_v.at[par], sem.at[par]).wait()
                pltpu.make_async_copy(q_hbm.at[col_g.at[gpar, jg]],
                                      qv_v.at[par], sem.at[par]).wait()

        zero16 = jnp.zeros((16,), jnp.float32)

        def zbody(r, carry):
            for v in range(8):
                rows_v[0, r, pl.ds(v * 16, 16)] = zero16
            return carry

        lax.fori_loop(0, CB, zbody, 0)
        base = s * NPT
        for k in range(NPT // CB):
            pltpu.sync_copy(rows_v.at[0], acc.at[pl.ds(base + k * CB, CB)])
        plsc.subcore_barrier()

        load_group(0, 0)
        for j0 in range(NB - 1):
            issue_gathers(0, j0, j0)

        def jbody(j, carry):
            jg = j % GC
            gidx = j // GC
            gpar = gidx % 2
            par = j % NB
            nj = j + NB - 1
            njg = nj % GC
            ngidx = nj // GC
            ngpar = ngidx % 2
            npar = nj % NB

            @pl.when(jnp.logical_and(nj < NCH, njg == 0))
            def _():
                load_group(ngidx, ngpar)

            @pl.when(nj < NCH)
            def _():
                @pl.when(nj >= NB)
                def _():
                    pltpu.make_async_copy(
                        rows_v.at[npar], acc.at[row_g.at[gpar, jg]],
                        ssem.at[npar]).wait()

                issue_gathers(ngpar, njg, npar)

            wait_gathers(gpar, jg, par)

            if first:
                for g in range(CB // 16):
                    sl = pl.ds(g * 16, 16)
                    sv = pv_v[par, sl] + qv_v[par, sl]
                    sv = jnp.where(sv > 0.0, sv, 0.2 * sv)
                    att = 1.0 / (1.0 + jnp.exp(-sv))
                    wv_g[gpar, jg, sl] = att * wv_g[gpar, jg, sl]

            @plsc.parallel_loop(0, CB // 16, unroll=4)
            def scale(k16):
                w16 = wv_g[gpar, jg, pl.ds(k16 * 16, 16)]
                for e in range(16):
                    wb = jnp.full((16,), w16[e])
                    r = k16 * 16 + e
                    for v in range(8):
                        sl = pl.ds(v * 16, 16)
                        rows_v[par, r, sl] = rows_v[par, r, sl] * wb

            pltpu.async_copy(rows_v.at[par], acc.at[row_g.at[gpar, jg]],
                             ssem.at[par], add=True)
            if first:
                @pl.when(jg == GC - 1)
                def _():
                    pltpu.sync_copy(wv_g.at[gpar],
                                    w_hbm.at[wid, pl.ds(gidx * GC, GC)])
            return carry

        lax.fori_loop(0, NCH, jbody, 0)
        for k in range(NB):
            pltpu.make_async_copy(rows_v.at[k], acc.at[row_g.at[0, 0]],
                                  ssem.at[k]).wait()
        plsc.subcore_barrier()
        pltpu.sync_copy(acc.at[pl.ds(base, NPT)], out_hbm.at[c, pl.ds(base, NPT)])

    return pl.kernel(body, mesh=mesh, out_type=out_type,
                     scratch_types=scratch,
                     compiler_params=pltpu.CompilerParams(
                         needs_layout_passes=False))


def kernel(x, edge_index, edge_values, W1, a1, W2, ln_gamma, ln_beta):
    row = edge_index[0]
    col = edge_index[1]
    pad = EP - E
    spread = (jnp.arange(pad, dtype=jnp.int32) * 37) % N
    rowp = jnp.concatenate([row, spread]).reshape(NTILES, NCH, CB)
    colp = jnp.concatenate([col, spread]).reshape(NTILES, NCH, CB)
    evp = jnp.concatenate([edge_values, jnp.zeros((pad,), jnp.float32)]).reshape(NTILES, NCH, CB)

    amat = jnp.zeros((D, D), jnp.float32)
    amat = amat.at[:, 0].set(a1[:D, 0]).at[:, 1].set(a1[D:, 0])

    h1, pq = _tc1(x, W1, amat)
    p = pq[:, 0]
    q = pq[:, 1]

    parts1, w = _make_spmm(True)(h1, rowp, colp, evp, p, q)

    h2 = _tc2(parts1[0], parts1[1], W2)

    (parts2,) = _make_spmm(False)(h2, rowp, colp, w)

    out = _tc3(parts2[0], parts2[1], x, ln_gamma.reshape(1, D),
               ln_beta.reshape(1, D))
    return out
